# Initial kernel scaffold; baseline (speedup 1.0000x reference)
import jax
import jax.numpy as jnp
from jax.experimental import pallas as pl


def kernel(data, length, table, w, b):
    raise NotImplementedError("write your pallas kernel here")


# speedup vs baseline: 12.3204x; 12.3204x over previous
import functools

import jax
import jax.numpy as jnp
from jax import lax
from jax.experimental import pallas as pl
from jax.experimental.pallas import tpu as pltpu
from jax.experimental.pallas import tpu_sc as plsc

VOCAB = 1000002
EMB = 32
BATCH = 16384
SEQ = 200

NUM_CORES = 2
NUM_SUBCORES = 16
NW = NUM_CORES * NUM_SUBCORES
SENT_PER_W = BATCH // NW
CHUNK = 128
NCHUNK = SENT_PER_W // CHUNK
ELEMS = CHUNK * SEQ
GROWS = ELEMS // 128

_TW_BLOCK = 8192
_TW_GRID = (VOCAB + _TW_BLOCK - 1) // _TW_BLOCK


def _tw_body(tab_ref, w_ref, o_ref):
    o_ref[...] = jnp.sum(tab_ref[...] * w_ref[...], axis=1, keepdims=True)


def _table_times_w(table, w):
    return pl.pallas_call(
        _tw_body,
        grid=(_TW_GRID,),
        in_specs=[
            pl.BlockSpec((_TW_BLOCK, EMB), lambda i: (i, 0)),
            pl.BlockSpec((1, EMB), lambda i: (0, 0)),
        ],
        out_specs=pl.BlockSpec((_TW_BLOCK, 1), lambda i: (i, 0)),
        out_shape=jax.ShapeDtypeStruct((VOCAB, 1), jnp.float32),
    )(table, w.reshape(1, EMB))


_mesh = plsc.VectorSubcoreMesh(core_axis_name="c", subcore_axis_name="s")


@functools.partial(
    pl.kernel,
    out_type=jax.ShapeDtypeStruct((BATCH,), jnp.float32),
    mesh=_mesh,
    compiler_params=pltpu.CompilerParams(needs_layout_passes=False),
    scratch_types=[
        pltpu.VMEM((GROWS, 128), jnp.int32),
        pltpu.VMEM((ELEMS,), jnp.float32),
        pltpu.VMEM((CHUNK,), jnp.int32),
        pltpu.VMEM((CHUNK,), jnp.float32),
        pltpu.VMEM((16,), jnp.float32),
        pltpu.SemaphoreType.DMA,
    ],
)
def _sc_pool(tw_hbm, data_hbm, len_hbm, b_hbm, out_hbm,
             idx_v, vals_v, len_v, out_v, b_v, gsem):
    cid = lax.axis_index("c")
    sid = lax.axis_index("s")
    wid = sid * NUM_CORES + cid

    pltpu.sync_copy(b_hbm, b_v)
    bvec = b_v[...]

    for ch in range(NCHUNK):
        col = wid * SENT_PER_W + ch * CHUNK
        row0 = (wid * NCHUNK + ch) * GROWS

        pltpu.sync_copy(data_hbm.at[pl.ds(row0, GROWS), :], idx_v)
        pltpu.sync_copy(len_hbm.at[pl.ds(col, CHUNK)], len_v)

        def _fire(r, carry):
            pltpu.async_copy(
                tw_hbm.at[idx_v.at[r]],
                vals_v.at[pl.ds(pl.multiple_of(r * 128, 128), 128)],
                gsem,
            )
            return carry

        lax.fori_loop(0, GROWS, _fire, 0)
        pltpu.make_async_copy(tw_hbm.at[pl.ds(0, ELEMS)], vals_v, gsem).wait()

        for g in range(CHUNK // 16):
            idx0 = lax.iota(jnp.int32, 16) * SEQ + (g * 16 * SEQ)

            def _cbody(c, carry):
                acc, idxv = carry
                for _ in range(8):
                    acc = acc + plsc.load_gather(vals_v, [idxv])
                    idxv = idxv + 1
                return (acc, idxv)

            acc, _ = lax.fori_loop(
                0, SEQ // 8, _cbody,
                (jnp.zeros((16,), jnp.float32), idx0))

            lenf = len_v[pl.ds(g * 16, 16)].astype(jnp.float32)
            x = acc / lenf + bvec
            out_v[pl.ds(g * 16, 16)] = 1.0 / (1.0 + jnp.exp(-x))

        pltpu.sync_copy(out_v, out_hbm.at[pl.ds(col, CHUNK)])


def kernel(data, length, table, w, b):
    tw = _table_times_w(table, w).reshape(-1)
    data_rows = data.reshape(-1, 128)
    b16 = jnp.broadcast_to(b.astype(jnp.float32), (16,))
    return _sc_pool(tw, data_rows, length.astype(jnp.int32), b16)


# speedup vs baseline: 40.0928x; 3.2542x over previous
import functools

import jax
import jax.numpy as jnp
from jax import lax
from jax.experimental import pallas as pl
from jax.experimental.pallas import tpu as pltpu
from jax.experimental.pallas import tpu_sc as plsc

VOCAB = 1000002
EMB = 32
BATCH = 16384
SEQ = 200

NUM_CORES = 2
NUM_SUBCORES = 16
NW = NUM_CORES * NUM_SUBCORES
SENT_PER_W = BATCH // NW
CHUNK = 128
NCHUNK = SENT_PER_W // CHUNK
ELEMS = CHUNK * SEQ

_TW_BLOCK = 8192
_TW_GRID = (VOCAB + _TW_BLOCK - 1) // _TW_BLOCK


def _tw_body(tabt_ref, w_ref, o_ref):
    o_ref[...] = jnp.sum(tabt_ref[...] * w_ref[...], axis=0, keepdims=True)


def _table_times_w(table_t, w):
    return pl.pallas_call(
        _tw_body,
        grid=(_TW_GRID,),
        in_specs=[
            pl.BlockSpec((EMB, _TW_BLOCK), lambda i: (0, i)),
            pl.BlockSpec((EMB, 1), lambda i: (0, 0)),
        ],
        out_specs=pl.BlockSpec((1, _TW_BLOCK), lambda i: (0, i)),
        out_shape=jax.ShapeDtypeStruct((1, VOCAB), jnp.float32),
    )(table_t, w.reshape(EMB, 1))


_mesh = plsc.VectorSubcoreMesh(core_axis_name="c", subcore_axis_name="s")


@functools.partial(
    pl.kernel,
    out_type=jax.ShapeDtypeStruct((BATCH,), jnp.float32),
    mesh=_mesh,
    compiler_params=pltpu.CompilerParams(needs_layout_passes=False),
    scratch_types=[
        pltpu.VMEM((SEQ, CHUNK), jnp.int32),
        pltpu.VMEM((ELEMS,), jnp.float32),
        pltpu.VMEM((CHUNK,), jnp.int32),
        pltpu.VMEM((CHUNK,), jnp.float32),
        pltpu.VMEM((16,), jnp.float32),
        pltpu.SemaphoreType.DMA,
    ],
)
def _sc_pool(tw_hbm, datat_hbm, len_hbm, b_hbm, out_hbm,
             idx_v, vals_v, len_v, out_v, b_v, gsem):
    cid = lax.axis_index("c")
    sid = lax.axis_index("s")
    wid = sid * NUM_CORES + cid

    pltpu.sync_copy(b_hbm, b_v)
    bvec = b_v[...]
    zero = jnp.zeros((16,), jnp.float32)

    for ch in range(NCHUNK):
        col = wid * SENT_PER_W + ch * CHUNK

        pltpu.sync_copy(datat_hbm.at[:, pl.ds(col, CHUNK)], idx_v)
        pltpu.sync_copy(len_hbm.at[pl.ds(col, CHUNK)], len_v)

        def _fire(r, carry):
            pltpu.async_copy(
                tw_hbm.at[idx_v.at[r]],
                vals_v.at[pl.ds(pl.multiple_of(r * CHUNK, CHUNK), CHUNK)],
                gsem,
            )
            return carry

        lax.fori_loop(0, SEQ, _fire, 0)
        pltpu.make_async_copy(tw_hbm.at[pl.ds(0, ELEMS)], vals_v, gsem).wait()

        def _srow(s, accs):
            base = s * CHUNK
            return tuple(
                accs[u] + vals_v[pl.ds(base + 16 * u, 16)] for u in range(8)
            )

        accs = lax.fori_loop(0, SEQ, _srow, (zero,) * 8)

        for u in range(8):
            lenf = len_v[pl.ds(u * 16, 16)].astype(jnp.float32)
            x = accs[u] / lenf + bvec
            out_v[pl.ds(u * 16, 16)] = 1.0 / (1.0 + jnp.exp(-x))

        pltpu.sync_copy(out_v, out_hbm.at[pl.ds(col, CHUNK)])


def kernel(data, length, table, w, b):
    tw = _table_times_w(table.T, w).reshape(-1)
    b16 = jnp.broadcast_to(b.astype(jnp.float32), (16,))
    return _sc_pool(tw, data.T, length.astype(jnp.int32), b16)


# speedup vs baseline: 71.3142x; 1.7787x over previous
import functools

import jax
import jax.numpy as jnp
from jax import lax
from jax.experimental import pallas as pl
from jax.experimental.pallas import tpu as pltpu
from jax.experimental.pallas import tpu_sc as plsc

VOCAB = 1000002
EMB = 32
BATCH = 16384
SEQ = 200

NUM_CORES = 2
NUM_SUBCORES = 16
NW = NUM_CORES * NUM_SUBCORES
SENT_PER_W = BATCH // NW
CHUNK = 128
NCHUNK = SENT_PER_W // CHUNK
ELEMS = CHUNK * SEQ

_TW_BLOCK = 8192
_TW_GRID = (VOCAB + _TW_BLOCK - 1) // _TW_BLOCK
VPAD = _TW_GRID * _TW_BLOCK
_STAGE = VPAD // NUM_SUBCORES


def _tw_body(tabt_ref, w_ref, o_ref):
    o_ref[...] = jnp.sum(tabt_ref[...] * w_ref[...], axis=0)


def _table_times_w(table_t, w):
    return pl.pallas_call(
        _tw_body,
        grid=(_TW_GRID,),
        in_specs=[
            pl.BlockSpec((EMB, _TW_BLOCK), lambda i: (0, i)),
            pl.BlockSpec((EMB, 1), lambda i: (0, 0)),
        ],
        out_specs=pl.BlockSpec((_TW_BLOCK,), lambda i: (i,)),
        out_shape=jax.ShapeDtypeStruct((VPAD,), jnp.float32),
    )(table_t, w.reshape(EMB, 1))


_mesh = plsc.VectorSubcoreMesh(core_axis_name="c", subcore_axis_name="s")


@functools.partial(
    pl.kernel,
    out_type=jax.ShapeDtypeStruct((BATCH,), jnp.float32),
    mesh=_mesh,
    compiler_params=pltpu.CompilerParams(needs_layout_passes=False),
    scratch_types=[
        pltpu.VMEM_SHARED((VPAD,), jnp.float32),
        pltpu.VMEM((SEQ, CHUNK), jnp.int32),
        pltpu.VMEM((ELEMS,), jnp.float32),
        pltpu.VMEM((CHUNK,), jnp.int32),
        pltpu.VMEM((CHUNK,), jnp.float32),
        pltpu.VMEM((16,), jnp.float32),
        pltpu.SemaphoreType.DMA,
    ],
)
def _sc_pool(tw_hbm, datat_hbm, len_hbm, b_hbm, out_hbm,
             tw_sp, idx_v, vals_v, len_v, out_v, b_v, gsem):
    cid = lax.axis_index("c")
    sid = lax.axis_index("s")
    wid = sid * NUM_CORES + cid

    off = sid * _STAGE
    pltpu.sync_copy(tw_hbm.at[pl.ds(off, _STAGE)], tw_sp.at[pl.ds(off, _STAGE)])
    pltpu.sync_copy(b_hbm, b_v)
    plsc.subcore_barrier()

    bvec = b_v[...]
    zero = jnp.zeros((16,), jnp.float32)

    for ch in range(NCHUNK):
        col = wid * SENT_PER_W + ch * CHUNK

        pltpu.sync_copy(datat_hbm.at[:, pl.ds(col, CHUNK)], idx_v)
        pltpu.sync_copy(len_hbm.at[pl.ds(col, CHUNK)], len_v)

        def _fire(r, carry):
            pltpu.async_copy(
                tw_sp.at[idx_v.at[r]],
                vals_v.at[pl.ds(pl.multiple_of(r * CHUNK, CHUNK), CHUNK)],
                gsem,
            )
            return carry

        lax.fori_loop(0, SEQ, _fire, 0)
        pltpu.make_async_copy(tw_hbm.at[pl.ds(0, ELEMS)], vals_v, gsem).wait()

        def _srow(s, accs):
            base = s * CHUNK
            return tuple(
                accs[u] + vals_v[pl.ds(base + 16 * u, 16)] for u in range(8)
            )

        accs = lax.fori_loop(0, SEQ, _srow, (zero,) * 8)

        for u in range(8):
            lenf = len_v[pl.ds(u * 16, 16)].astype(jnp.float32)
            x = accs[u] / lenf + bvec
            out_v[pl.ds(u * 16, 16)] = 1.0 / (1.0 + jnp.exp(-x))

        pltpu.sync_copy(out_v, out_hbm.at[pl.ds(col, CHUNK)])


def kernel(data, length, table, w, b):
    tw = _table_times_w(table.T, w)
    b16 = jnp.broadcast_to(b.astype(jnp.float32), (16,))
    return _sc_pool(tw, data.T, length.astype(jnp.int32), b16)


# speedup vs baseline: 98.8073x; 1.3855x over previous
import functools

import jax
import jax.numpy as jnp
from jax import lax
from jax.experimental import pallas as pl
from jax.experimental.pallas import tpu as pltpu
from jax.experimental.pallas import tpu_sc as plsc

VOCAB = 1000002
EMB = 32
BATCH = 16384
SEQ = 200

NUM_CORES = 2
NUM_SUBCORES = 16
NW = NUM_CORES * NUM_SUBCORES
SENT_PER_W = BATCH // NW
CHUNK = 128
NCHUNK = SENT_PER_W // CHUNK
ELEMS = CHUNK * SEQ

_TW_BLOCK = 32768
_TW_GRID = (VOCAB + _TW_BLOCK - 1) // _TW_BLOCK
VPAD = _TW_GRID * _TW_BLOCK
_STAGE = VPAD // NUM_SUBCORES


def _tw_body(tabt_ref, w_ref, o_ref):
    o_ref[...] = jnp.sum(tabt_ref[...] * w_ref[...], axis=0)


def _table_times_w(table_t, w):
    return pl.pallas_call(
        _tw_body,
        grid=(_TW_GRID,),
        in_specs=[
            pl.BlockSpec((EMB, _TW_BLOCK), lambda i: (0, i)),
            pl.BlockSpec((EMB, 1), lambda i: (0, 0)),
        ],
        out_specs=pl.BlockSpec((_TW_BLOCK,), lambda i: (i,)),
        out_shape=jax.ShapeDtypeStruct((VPAD,), jnp.float32),
    )(table_t, w.reshape(EMB, 1))


_mesh = plsc.VectorSubcoreMesh(core_axis_name="c", subcore_axis_name="s")


@functools.partial(
    pl.kernel,
    out_type=jax.ShapeDtypeStruct((BATCH,), jnp.float32),
    mesh=_mesh,
    compiler_params=pltpu.CompilerParams(needs_layout_passes=False),
    scratch_types=[
        pltpu.VMEM_SHARED((VPAD,), jnp.float32),
        pltpu.VMEM((SEQ, CHUNK), jnp.int32),
        pltpu.VMEM((ELEMS,), jnp.float32),
        pltpu.VMEM((CHUNK,), jnp.int32),
        pltpu.VMEM((CHUNK,), jnp.float32),
        pltpu.VMEM((16,), jnp.float32),
        pltpu.SemaphoreType.DMA,
    ],
)
def _sc_pool(tw_hbm, datat_hbm, len_hbm, b_hbm, out_hbm,
             tw_sp, idx_v, vals_v, len_v, out_v, b_v, gsem):
    cid = lax.axis_index("c")
    sid = lax.axis_index("s")
    wid = sid * NUM_CORES + cid

    off = sid * _STAGE
    pltpu.sync_copy(tw_hbm.at[pl.ds(off, _STAGE)], tw_sp.at[pl.ds(off, _STAGE)])
    pltpu.sync_copy(b_hbm, b_v)
    plsc.subcore_barrier()

    bvec = b_v[...]
    zero = jnp.zeros((16,), jnp.float32)

    for ch in range(NCHUNK):
        col = wid * SENT_PER_W + ch * CHUNK

        pltpu.sync_copy(datat_hbm.at[:, pl.ds(col, CHUNK)], idx_v)
        pltpu.sync_copy(len_hbm.at[pl.ds(col, CHUNK)], len_v)

        def _fire(r, carry):
            pltpu.async_copy(
                tw_sp.at[idx_v.at[r]],
                vals_v.at[pl.ds(pl.multiple_of(r * CHUNK, CHUNK), CHUNK)],
                gsem,
            )
            return carry

        lax.fori_loop(0, SEQ, _fire, 0)
        pltpu.make_async_copy(tw_hbm.at[pl.ds(0, ELEMS)], vals_v, gsem).wait()

        def _srow(s, accs):
            base = s * CHUNK
            return tuple(
                accs[u] + vals_v[pl.ds(base + 16 * u, 16)] for u in range(8)
            )

        accs = lax.fori_loop(0, SEQ, _srow, (zero,) * 8)

        for u in range(8):
            lenf = len_v[pl.ds(u * 16, 16)].astype(jnp.float32)
            x = accs[u] / lenf + bvec
            out_v[pl.ds(u * 16, 16)] = 1.0 / (1.0 + jnp.exp(-x))

        pltpu.sync_copy(out_v, out_hbm.at[pl.ds(col, CHUNK)])


def kernel(data, length, table, w, b):
    tw = _table_times_w(table.T, w)
    b16 = jnp.broadcast_to(b.astype(jnp.float32), (16,))
    return _sc_pool(tw, data.T, length.astype(jnp.int32), b16)


# speedup vs baseline: 104.5852x; 1.0585x over previous
import functools

import jax
import jax.numpy as jnp
from jax import lax
from jax.experimental import pallas as pl
from jax.experimental.pallas import tpu as pltpu
from jax.experimental.pallas import tpu_sc as plsc

VOCAB = 1000002
EMB = 32
BATCH = 16384
SEQ = 200

NUM_CORES = 2
NUM_SUBCORES = 16
NW = NUM_CORES * NUM_SUBCORES
SENT_PER_W = BATCH // NW
CHUNK = 128
NCHUNK = SENT_PER_W // CHUNK
ELEMS = CHUNK * SEQ

_TW_BLOCK = 65536
_TW_GRID = (VOCAB + _TW_BLOCK - 1) // _TW_BLOCK
VPAD = _TW_GRID * _TW_BLOCK
_STAGE = VPAD // NUM_SUBCORES


def _tw_body(tabt_ref, w_ref, o_ref):
    o_ref[...] = jnp.sum(tabt_ref[...] * w_ref[...], axis=0)


def _table_times_w(table_t, w):
    return pl.pallas_call(
        _tw_body,
        grid=(_TW_GRID,),
        in_specs=[
            pl.BlockSpec((EMB, _TW_BLOCK), lambda i: (0, i)),
            pl.BlockSpec((EMB, 1), lambda i: (0, 0)),
        ],
        out_specs=pl.BlockSpec((_TW_BLOCK,), lambda i: (i,)),
        out_shape=jax.ShapeDtypeStruct((VPAD,), jnp.float32),
    )(table_t, w.reshape(EMB, 1))


_mesh = plsc.VectorSubcoreMesh(core_axis_name="c", subcore_axis_name="s")


@functools.partial(
    pl.kernel,
    out_type=jax.ShapeDtypeStruct((BATCH,), jnp.float32),
    mesh=_mesh,
    compiler_params=pltpu.CompilerParams(needs_layout_passes=False),
    scratch_types=[
        pltpu.VMEM_SHARED((VPAD,), jnp.float32),
        pltpu.VMEM((SEQ, CHUNK), jnp.int32),
        pltpu.VMEM((ELEMS,), jnp.float32),
        pltpu.VMEM((CHUNK,), jnp.int32),
        pltpu.VMEM((CHUNK,), jnp.float32),
        pltpu.VMEM((16,), jnp.float32),
        pltpu.SemaphoreType.DMA,
    ],
)
def _sc_pool(tw_hbm, datat_hbm, len_hbm, b_hbm, out_hbm,
             tw_sp, idx_v, vals_v, len_v, out_v, b_v, gsem):
    cid = lax.axis_index("c")
    sid = lax.axis_index("s")
    wid = sid * NUM_CORES + cid

    off = sid * _STAGE
    pltpu.sync_copy(tw_hbm.at[pl.ds(off, _STAGE)], tw_sp.at[pl.ds(off, _STAGE)])
    pltpu.sync_copy(b_hbm, b_v)
    plsc.subcore_barrier()

    bvec = b_v[...]
    zero = jnp.zeros((16,), jnp.float32)

    for ch in range(NCHUNK):
        col = wid * SENT_PER_W + ch * CHUNK

        pltpu.sync_copy(datat_hbm.at[:, pl.ds(col, CHUNK)], idx_v)
        pltpu.sync_copy(len_hbm.at[pl.ds(col, CHUNK)], len_v)

        def _fire(r, carry):
            pltpu.async_copy(
                tw_sp.at[idx_v.at[r]],
                vals_v.at[pl.ds(pl.multiple_of(r * CHUNK, CHUNK), CHUNK)],
                gsem,
            )
            return carry

        lax.fori_loop(0, SEQ, _fire, 0)
        pltpu.make_async_copy(tw_hbm.at[pl.ds(0, ELEMS)], vals_v, gsem).wait()

        def _srow(s, accs):
            base = s * CHUNK
            return tuple(
                accs[u] + vals_v[pl.ds(base + 16 * u, 16)] for u in range(8)
            )

        accs = lax.fori_loop(0, SEQ, _srow, (zero,) * 8)

        for u in range(8):
            lenf = len_v[pl.ds(u * 16, 16)].astype(jnp.float32)
            x = accs[u] / lenf + bvec
            out_v[pl.ds(u * 16, 16)] = 1.0 / (1.0 + jnp.exp(-x))

        pltpu.sync_copy(out_v, out_hbm.at[pl.ds(col, CHUNK)])


def kernel(data, length, table, w, b):
    tw = _table_times_w(table.T, w)
    b16 = jnp.broadcast_to(b.astype(jnp.float32), (16,))
    return _sc_pool(tw, data.T, length.astype(jnp.int32), b16)


# speedup vs baseline: 108.7570x; 1.0399x over previous
import functools

import jax
import jax.numpy as jnp
from jax import lax
from jax.experimental import pallas as pl
from jax.experimental.pallas import tpu as pltpu
from jax.experimental.pallas import tpu_sc as plsc

VOCAB = 1000002
EMB = 32
BATCH = 16384
SEQ = 200

NUM_CORES = 2
NUM_SUBCORES = 16
NW = NUM_CORES * NUM_SUBCORES
SENT_PER_W = BATCH // NW
CHUNK = 128
NCHUNK = SENT_PER_W // CHUNK
_H0 = 96
_H1 = SEQ - _H0

_TW_BLOCK = 65536
_TW_GRID = (VOCAB + _TW_BLOCK - 1) // _TW_BLOCK
VPAD = _TW_GRID * _TW_BLOCK
_STAGE = VPAD // NUM_SUBCORES


def _tw_body(tabt_ref, w_ref, o_ref):
    o_ref[...] = jnp.sum(tabt_ref[...] * w_ref[...], axis=0)


def _table_times_w(table_t, w):
    return pl.pallas_call(
        _tw_body,
        grid=(_TW_GRID,),
        in_specs=[
            pl.BlockSpec((EMB, _TW_BLOCK), lambda i: (0, i)),
            pl.BlockSpec((EMB, 1), lambda i: (0, 0)),
        ],
        out_specs=pl.BlockSpec((_TW_BLOCK,), lambda i: (i,)),
        out_shape=jax.ShapeDtypeStruct((VPAD,), jnp.float32),
    )(table_t, w.reshape(EMB, 1))


_mesh = plsc.VectorSubcoreMesh(core_axis_name="c", subcore_axis_name="s")


@functools.partial(
    pl.kernel,
    out_type=jax.ShapeDtypeStruct((BATCH,), jnp.float32),
    mesh=_mesh,
    compiler_params=pltpu.CompilerParams(needs_layout_passes=False),
    scratch_types=[
        pltpu.VMEM_SHARED((VPAD,), jnp.float32),
        pltpu.VMEM((2, _H1, CHUNK), jnp.int32),
        pltpu.VMEM((2, _H1 * CHUNK), jnp.float32),
        pltpu.VMEM((CHUNK,), jnp.int32),
        pltpu.VMEM((CHUNK,), jnp.float32),
        pltpu.VMEM((16,), jnp.float32),
        pltpu.SemaphoreType.DMA,
        pltpu.SemaphoreType.DMA,
    ],
)
def _sc_pool(tw_hbm, datat_hbm, len_hbm, b_hbm, out_hbm,
             tw_sp, idx_v, vals_v, len_v, out_v, b_v, sem0, sem1):
    cid = lax.axis_index("c")
    sid = lax.axis_index("s")
    wid = sid * NUM_CORES + cid
    sems = (sem0, sem1)
    col0 = wid * SENT_PER_W

    off = sid * _STAGE
    pltpu.sync_copy(tw_hbm.at[pl.ds(off, _STAGE)], tw_sp.at[pl.ds(off, _STAGE)])
    pltpu.sync_copy(b_hbm, b_v)
    plsc.subcore_barrier()

    bvec = b_v[...]
    zero = jnp.zeros((16,), jnp.float32)

    units = [(ch, h) for ch in range(NCHUNK) for h in range(2)]

    def _load_and_fire(i):
        buf = i % 2
        ch, h = units[i]
        r0, nr = (0, _H0) if h == 0 else (_H0, _H1)
        pltpu.sync_copy(
            datat_hbm.at[pl.ds(r0, nr), pl.ds(col0 + ch * CHUNK, CHUNK)],
            idx_v.at[buf, pl.ds(0, nr)],
        )

        def body(r, carry):
            pltpu.async_copy(
                tw_sp.at[idx_v.at[buf, r]],
                vals_v.at[buf, pl.ds(pl.multiple_of(r * CHUNK, CHUNK), CHUNK)],
                sems[buf],
            )
            return carry

        lax.fori_loop(0, nr, body, 0)

    _load_and_fire(0)
    accs = (zero,) * 8

    for i in range(len(units)):
        buf = i % 2
        ch, h = units[i]
        nr = _H0 if h == 0 else _H1

        if i + 1 < len(units):
            _load_and_fire(i + 1)

        pltpu.make_async_copy(
            tw_hbm.at[pl.ds(0, nr * CHUNK)],
            vals_v.at[buf, pl.ds(0, nr * CHUNK)],
            sems[buf],
        ).wait()

        def _srow(s, a):
            base = s * CHUNK
            return tuple(
                a[u] + vals_v[buf, pl.ds(base + 16 * u, 16)] for u in range(8)
            )

        accs = lax.fori_loop(0, nr, _srow, accs)

        if h == 1:
            col = col0 + ch * CHUNK
            pltpu.sync_copy(len_hbm.at[pl.ds(col, CHUNK)], len_v)
            for u in range(8):
                lenf = len_v[pl.ds(u * 16, 16)].astype(jnp.float32)
                x = accs[u] / lenf + bvec
                out_v[pl.ds(u * 16, 16)] = 1.0 / (1.0 + jnp.exp(-x))
            pltpu.sync_copy(out_v, out_hbm.at[pl.ds(col, CHUNK)])
            accs = (zero,) * 8


def kernel(data, length, table, w, b):
    tw = _table_times_w(table.T, w)
    b16 = jnp.broadcast_to(b.astype(jnp.float32), (16,))
    return _sc_pool(tw, data.T, length.astype(jnp.int32), b16)


# speedup vs baseline: 109.4253x; 1.0061x over previous
import functools

import jax
import jax.numpy as jnp
from jax import lax
from jax.experimental import pallas as pl
from jax.experimental.pallas import tpu as pltpu
from jax.experimental.pallas import tpu_sc as plsc

VOCAB = 1000002
EMB = 32
BATCH = 16384
SEQ = 200

NUM_CORES = 2
NUM_SUBCORES = 16
NW = NUM_CORES * NUM_SUBCORES
SENT_PER_W = BATCH // NW
CHUNK = 128
NCHUNK = SENT_PER_W // CHUNK
_H0 = 96
_H1 = SEQ - _H0

_TW_BLOCK = 131072
_TW_GRID = (VOCAB + _TW_BLOCK - 1) // _TW_BLOCK
VPAD = _TW_GRID * _TW_BLOCK
_STAGE = VPAD // NUM_SUBCORES


def _tw_body(tabt_ref, w_ref, o_ref):
    o_ref[...] = jnp.sum(tabt_ref[...] * w_ref[...], axis=0)


def _table_times_w(table_t, w):
    return pl.pallas_call(
        _tw_body,
        grid=(_TW_GRID,),
        in_specs=[
            pl.BlockSpec((EMB, _TW_BLOCK), lambda i: (0, i)),
            pl.BlockSpec((EMB, 1), lambda i: (0, 0)),
        ],
        out_specs=pl.BlockSpec((_TW_BLOCK,), lambda i: (i,)),
        out_shape=jax.ShapeDtypeStruct((VPAD,), jnp.float32),
    )(table_t, w.reshape(EMB, 1))


_mesh = plsc.VectorSubcoreMesh(core_axis_name="c", subcore_axis_name="s")


@functools.partial(
    pl.kernel,
    out_type=jax.ShapeDtypeStruct((BATCH,), jnp.float32),
    mesh=_mesh,
    compiler_params=pltpu.CompilerParams(needs_layout_passes=False),
    scratch_types=[
        pltpu.VMEM_SHARED((VPAD,), jnp.float32),
        pltpu.VMEM((2, _H1, CHUNK), jnp.int32),
        pltpu.VMEM((2, _H1 * CHUNK), jnp.float32),
        pltpu.VMEM((CHUNK,), jnp.int32),
        pltpu.VMEM((CHUNK,), jnp.float32),
        pltpu.VMEM((16,), jnp.float32),
        pltpu.SemaphoreType.DMA,
        pltpu.SemaphoreType.DMA,
    ],
)
def _sc_pool(tw_hbm, datat_hbm, len_hbm, b_hbm, out_hbm,
             tw_sp, idx_v, vals_v, len_v, out_v, b_v, sem0, sem1):
    cid = lax.axis_index("c")
    sid = lax.axis_index("s")
    wid = sid * NUM_CORES + cid
    sems = (sem0, sem1)
    col0 = wid * SENT_PER_W

    off = sid * _STAGE
    pltpu.sync_copy(tw_hbm.at[pl.ds(off, _STAGE)], tw_sp.at[pl.ds(off, _STAGE)])
    pltpu.sync_copy(b_hbm, b_v)
    plsc.subcore_barrier()

    bvec = b_v[...]
    zero = jnp.zeros((16,), jnp.float32)

    units = [(ch, h) for ch in range(NCHUNK) for h in range(2)]

    def _load_and_fire(i):
        buf = i % 2
        ch, h = units[i]
        r0, nr = (0, _H0) if h == 0 else (_H0, _H1)
        pltpu.sync_copy(
            datat_hbm.at[pl.ds(r0, nr), pl.ds(col0 + ch * CHUNK, CHUNK)],
            idx_v.at[buf, pl.ds(0, nr)],
        )

        def body(r, carry):
            pltpu.async_copy(
                tw_sp.at[idx_v.at[buf, r]],
                vals_v.at[buf, pl.ds(pl.multiple_of(r * CHUNK, CHUNK), CHUNK)],
                sems[buf],
            )
            return carry

        lax.fori_loop(0, nr, body, 0)

    _load_and_fire(0)
    accs = (zero,) * 8

    for i in range(len(units)):
        buf = i % 2
        ch, h = units[i]
        nr = _H0 if h == 0 else _H1

        if i + 1 < len(units):
            _load_and_fire(i + 1)

        pltpu.make_async_copy(
            tw_hbm.at[pl.ds(0, nr * CHUNK)],
            vals_v.at[buf, pl.ds(0, nr * CHUNK)],
            sems[buf],
        ).wait()

        def _srow(s, a):
            base = s * CHUNK
            return tuple(
                a[u] + vals_v[buf, pl.ds(base + 16 * u, 16)] for u in range(8)
            )

        accs = lax.fori_loop(0, nr, _srow, accs)

        if h == 1:
            col = col0 + ch * CHUNK
            pltpu.sync_copy(len_hbm.at[pl.ds(col, CHUNK)], len_v)
            for u in range(8):
                lenf = len_v[pl.ds(u * 16, 16)].astype(jnp.float32)
                x = accs[u] / lenf + bvec
                out_v[pl.ds(u * 16, 16)] = 1.0 / (1.0 + jnp.exp(-x))
            pltpu.sync_copy(out_v, out_hbm.at[pl.ds(col, CHUNK)])
            accs = (zero,) * 8


def kernel(data, length, table, w, b):
    tw = _table_times_w(table.T, w)
    b16 = jnp.broadcast_to(b.astype(jnp.float32), (16,))
    return _sc_pool(tw, data.T, length.astype(jnp.int32), b16)
